# Initial kernel scaffold; baseline (speedup 1.0000x reference)
import jax
import jax.numpy as jnp
from jax.experimental import pallas as pl


def kernel(x, W_q, ln_scale, ln_bias, keys, values):
    raise NotImplementedError("write your pallas kernel here")


# speedup vs baseline: 7.1504x; 7.1504x over previous
import functools

import jax
import jax.numpy as jnp
from jax import lax
from jax.experimental import pallas as pl
from jax.experimental.pallas import tpu as pltpu
from jax.experimental.pallas import tpu_sc as plsc

DIM = 1024
HEADS = 4
NKEYS = 256
K = 16
DHALF = 128
T = 2048
KH = HEADS * K
TBLK = 256
NBLK = T // TBLK
NEG = float("-inf")


def _topk16(d, iota):
    vals, idxs = [], []
    for _ in range(K):
        m = jnp.max(d, axis=1, keepdims=True)
        eq = d == m
        idx = jnp.min(jnp.where(eq, iota, 1 << 20), axis=1, keepdims=True)
        vals.append(m)
        idxs.append(idx)
        d = jnp.where(iota == idx, NEG, d)
    return jnp.concatenate(vals, axis=1), jnp.concatenate(idxs, axis=1)


def _tc_body(x_ref, wq_ref, lns_ref, lnb_ref, kmat_ref, vi_ref, attn_ref):
    xb = x_ref[...]
    q = jnp.dot(xb, wq_ref[...], preferred_element_type=jnp.float32,
                precision=lax.Precision.DEFAULT)
    mu = jnp.mean(q, axis=1, keepdims=True)
    var = jnp.mean((q - mu) ** 2, axis=1, keepdims=True)
    q = (q - mu) * lax.rsqrt(var + 1e-6) * lns_ref[...] + lnb_ref[...]

    dots_list = []
    for p in range(2):
        for h in range(HEADS):
            qph = q[:, p * (HEADS * DHALF) + h * DHALF:
                    p * (HEADS * DHALF) + (h + 1) * DHALF]
            dots_list.append(jnp.dot(qph, kmat_ref[p * HEADS + h],
                                     preferred_element_type=jnp.float32,
                                     precision=lax.Precision.DEFAULT))
    d1 = jnp.concatenate(dots_list, axis=0)
    iota_a = lax.broadcasted_iota(jnp.int32, (2 * HEADS * TBLK, NKEYS), 1)
    sv, si = _topk16(d1, iota_a)

    comb_list = []
    for h in range(HEADS):
        s1h = sv[h * TBLK:(h + 1) * TBLK]
        s2h = sv[(HEADS + h) * TBLK:(HEADS + h + 1) * TBLK]
        comb_list.append((s1h[:, :, None] + s2h[:, None, :]).reshape(TBLK, K * K))
    comb = jnp.concatenate(comb_list, axis=0)
    iota_b = lax.broadcasted_iota(jnp.int32, (HEADS * TBLK, K * K), 1)
    fv, fc = _topk16(comb, iota_b)

    a = fc >> 4
    b = fc & 15
    i1all = jnp.concatenate([si[h * TBLK:(h + 1) * TBLK] for h in range(HEADS)], axis=0)
    i2all = jnp.concatenate([si[(HEADS + h) * TBLK:(HEADS + h + 1) * TBLK]
                             for h in range(HEADS)], axis=0)
    iota16_3 = lax.broadcasted_iota(jnp.int32, (HEADS * TBLK, K, K), 2)
    i1sel = jnp.sum(jnp.where(a[:, :, None] == iota16_3, i1all[:, None, :], 0), axis=2)
    i2sel = jnp.sum(jnp.where(b[:, :, None] == iota16_3, i2all[:, None, :], 0), axis=2)
    vi = i1sel * NKEYS + i2sel
    mx = jnp.max(fv, axis=1, keepdims=True)
    e = jnp.exp(fv - mx)
    attn = e / jnp.sum(e, axis=1, keepdims=True)
    for h in range(HEADS):
        vi_ref[:, h * K:(h + 1) * K] = vi[h * TBLK:(h + 1) * TBLK]
        attn_ref[:, h * K:(h + 1) * K] = attn[h * TBLK:(h + 1) * TBLK]


def _tc_route(x2d, W_q, lns2d, lnb2d, kmat):
    return pl.pallas_call(
        _tc_body,
        grid=(NBLK,),
        in_specs=[
            pl.BlockSpec((TBLK, DIM), lambda i: (i, 0)),
            pl.BlockSpec((DIM, DIM), lambda i: (0, 0)),
            pl.BlockSpec((1, DIM), lambda i: (0, 0)),
            pl.BlockSpec((1, DIM), lambda i: (0, 0)),
            pl.BlockSpec((2 * HEADS, DHALF, NKEYS), lambda i: (0, 0, 0)),
        ],
        out_specs=[
            pl.BlockSpec((TBLK, KH), lambda i: (i, 0)),
            pl.BlockSpec((TBLK, KH), lambda i: (i, 0)),
        ],
        out_shape=[
            jax.ShapeDtypeStruct((T, KH), jnp.int32),
            jax.ShapeDtypeStruct((T, KH), jnp.float32),
        ],
    )(x2d, W_q, lns2d, lnb2d, kmat)


_NC = 2
_NS = 16
_NW = _NC * _NS
_TPW = T // _NW
_U = 8


def _sc_bag_body(vi_hbm, wrep_hbm, table_hbm, out_hbm,
                 idx_v, w_v, rows_v, acc_v, sem):
    wid = lax.axis_index("s") * _NC + lax.axis_index("c")
    base = wid * _TPW
    pltpu.sync_copy(vi_hbm.at[pl.ds(base, _TPW)], idx_v)

    def token_body(tl, carry):
        pltpu.sync_copy(wrep_hbm.at[base + tl], w_v)
        pltpu.async_copy(table_hbm.at[idx_v.at[tl]], rows_v, sem).wait()

        def dgroup(g, carry2):
            def rbody(r, accs):
                wv = w_v[r]
                return tuple(
                    accs[u] + wv * rows_v[r, pl.ds((g * _U + u) * 16, 16)]
                    for u in range(_U))
            accs = lax.fori_loop(
                0, KH, rbody,
                tuple(jnp.zeros((16,), jnp.float32) for _ in range(_U)))
            for u in range(_U):
                acc_v[pl.ds((g * _U + u) * 16, 16)] = accs[u]
            return carry2
        lax.fori_loop(0, DIM // (16 * _U), dgroup, 0)
        pltpu.sync_copy(acc_v, out_hbm.at[base + tl])
        return carry
    lax.fori_loop(0, _TPW, token_body, 0)


@functools.lru_cache(maxsize=1)
def _get_sc_bag():
    return functools.partial(
        pl.kernel,
        mesh=plsc.VectorSubcoreMesh(core_axis_name="c", subcore_axis_name="s"),
        out_type=jax.ShapeDtypeStruct((T, DIM), jnp.float32),
        scratch_types=[
            pltpu.VMEM((_TPW, KH), jnp.int32),
            pltpu.VMEM((KH, 16), jnp.float32),
            pltpu.VMEM((KH, DIM), jnp.float32),
            pltpu.VMEM((DIM,), jnp.float32),
            pltpu.SemaphoreType.DMA,
        ],
    )(_sc_bag_body)


def kernel(x, W_q, ln_scale, ln_bias, keys, values):
    t, b, e = x.shape
    x2d = x.reshape(T, DIM)
    lns2d = ln_scale.reshape(1, DIM)
    lnb2d = ln_bias.reshape(1, DIM)
    kmat = jnp.transpose(keys, (2, 0, 3, 1)).reshape(2 * HEADS, DHALF, NKEYS)
    vi, attn = _tc_route(x2d, W_q, lns2d, lnb2d, kmat)
    wrep = jnp.broadcast_to(attn[:, :, None], (T, KH, 16)) + jnp.zeros(
        (T, KH, 16), jnp.float32)
    out = _get_sc_bag()(vi, wrep, values)
    return out.reshape(t, b, e)


# speedup vs baseline: 8.7497x; 1.2237x over previous
import functools

import jax
import jax.numpy as jnp
from jax import lax
from jax.experimental import pallas as pl
from jax.experimental.pallas import tpu as pltpu
from jax.experimental.pallas import tpu_sc as plsc

DIM = 1024
HEADS = 4
NKEYS = 256
K = 16
DHALF = 128
T = 2048
KH = HEADS * K
TBLK = 256
NBLK = T // TBLK
NEG = float("-inf")


def _topk16_packed(d, idx_lane):
    di = lax.bitcast_convert_type(d, jnp.int32)
    dp = lax.bitcast_convert_type((di & jnp.int32(-256)) | idx_lane, jnp.float32)
    vals = []
    for _ in range(K):
        m = jnp.max(dp, axis=1, keepdims=True)
        vals.append(m)
        dp = jnp.where(dp == m, NEG, dp)
    v = jnp.concatenate(vals, axis=1)
    idx = lax.bitcast_convert_type(v, jnp.int32) & 255
    return v, idx


def _tc_body(x_ref, wq_ref, lns_ref, lnb_ref, kmat_ref, rep_ref, vi_ref, wrep_ref):
    xb = x_ref[...]
    q = jnp.dot(xb, wq_ref[...], preferred_element_type=jnp.float32,
                precision=lax.Precision.DEFAULT)
    mu = jnp.mean(q, axis=1, keepdims=True)
    var = jnp.mean((q - mu) ** 2, axis=1, keepdims=True)
    q = (q - mu) * lax.rsqrt(var + 1e-6) * lns_ref[...] + lnb_ref[...]

    dots_list = []
    for p in range(2):
        for h in range(HEADS):
            qph = q[:, p * (HEADS * DHALF) + h * DHALF:
                    p * (HEADS * DHALF) + (h + 1) * DHALF]
            dots_list.append(jnp.dot(qph, kmat_ref[p * HEADS + h],
                                     preferred_element_type=jnp.float32,
                                     precision=lax.Precision.DEFAULT))
    d1 = jnp.concatenate(dots_list, axis=0)
    iota_a = lax.broadcasted_iota(jnp.int32, (2 * HEADS * TBLK, NKEYS), 1)
    sv, si = _topk16_packed(d1, iota_a)

    comb_list = []
    for h in range(HEADS):
        s1h = sv[h * TBLK:(h + 1) * TBLK]
        s2h = sv[(HEADS + h) * TBLK:(HEADS + h + 1) * TBLK]
        comb_list.append((s1h[:, :, None] + s2h[:, None, :]).reshape(TBLK, K * K))
    comb = jnp.concatenate(comb_list, axis=0)
    iota_b = lax.broadcasted_iota(jnp.int32, (HEADS * TBLK, K * K), 1)
    fv, fc = _topk16_packed(comb, iota_b)

    a = fc >> 4
    b = fc & 15
    i1all = jnp.concatenate([si[h * TBLK:(h + 1) * TBLK] for h in range(HEADS)], axis=0)
    i2all = jnp.concatenate([si[(HEADS + h) * TBLK:(HEADS + h + 1) * TBLK]
                             for h in range(HEADS)], axis=0)
    iota16_3 = lax.broadcasted_iota(jnp.int32, (HEADS * TBLK, K, K), 2)
    i1sel = jnp.sum(jnp.where(a[:, :, None] == iota16_3, i1all[:, None, :], 0), axis=2)
    i2sel = jnp.sum(jnp.where(b[:, :, None] == iota16_3, i2all[:, None, :], 0), axis=2)
    vi = i1sel * NKEYS + i2sel
    mx = jnp.max(fv, axis=1, keepdims=True)
    e = jnp.exp(fv - mx)
    attn = e / jnp.sum(e, axis=1, keepdims=True)
    for h in range(HEADS):
        vi_ref[:, h * K:(h + 1) * K] = vi[h * TBLK:(h + 1) * TBLK]
        wrep_ref[:, h * (K * 16):(h + 1) * (K * 16)] = jnp.dot(
            attn[h * TBLK:(h + 1) * TBLK], rep_ref[...],
            preferred_element_type=jnp.float32)


def _tc_route(x2d, W_q, lns2d, lnb2d, kmat, rep):
    return pl.pallas_call(
        _tc_body,
        grid=(NBLK,),
        in_specs=[
            pl.BlockSpec((TBLK, DIM), lambda i: (i, 0)),
            pl.BlockSpec((DIM, DIM), lambda i: (0, 0)),
            pl.BlockSpec((1, DIM), lambda i: (0, 0)),
            pl.BlockSpec((1, DIM), lambda i: (0, 0)),
            pl.BlockSpec((2 * HEADS, DHALF, NKEYS), lambda i: (0, 0, 0)),
            pl.BlockSpec((K, K * 16), lambda i: (0, 0)),
        ],
        out_specs=[
            pl.BlockSpec((TBLK, KH), lambda i: (i, 0)),
            pl.BlockSpec((TBLK, KH * 16), lambda i: (i, 0)),
        ],
        out_shape=[
            jax.ShapeDtypeStruct((T, KH), jnp.int32),
            jax.ShapeDtypeStruct((T, KH * 16), jnp.float32),
        ],
    )(x2d, W_q, lns2d, lnb2d, kmat, rep)


_NC = 2
_NS = 16
_NW = _NC * _NS
_TPW = T // _NW
_U = 8


def _sc_bag_body(vi_hbm, wrep_hbm, table_hbm, out_hbm,
                 idx_v, w_v, rows_v, acc_v, sem):
    wid = lax.axis_index("s") * _NC + lax.axis_index("c")
    base = wid * _TPW
    pltpu.sync_copy(vi_hbm.at[pl.ds(base, _TPW)], idx_v)

    def token_body(tl, carry):
        pltpu.sync_copy(wrep_hbm.at[base + tl], w_v)
        pltpu.async_copy(table_hbm.at[idx_v.at[tl]], rows_v, sem).wait()

        def dgroup(g, carry2):
            def rbody(r, accs):
                wv = w_v[r]
                return tuple(
                    accs[u] + wv * rows_v[r, pl.ds((g * _U + u) * 16, 16)]
                    for u in range(_U))
            accs = lax.fori_loop(
                0, KH, rbody,
                tuple(jnp.zeros((16,), jnp.float32) for _ in range(_U)))
            for u in range(_U):
                acc_v[pl.ds((g * _U + u) * 16, 16)] = accs[u]
            return carry2
        lax.fori_loop(0, DIM // (16 * _U), dgroup, 0)
        pltpu.sync_copy(acc_v, out_hbm.at[base + tl])
        return carry
    lax.fori_loop(0, _TPW, token_body, 0)


@functools.lru_cache(maxsize=1)
def _get_sc_bag():
    return functools.partial(
        pl.kernel,
        mesh=plsc.VectorSubcoreMesh(core_axis_name="c", subcore_axis_name="s"),
        out_type=jax.ShapeDtypeStruct((T, DIM), jnp.float32),
        scratch_types=[
            pltpu.VMEM((_TPW, KH), jnp.int32),
            pltpu.VMEM((KH, 16), jnp.float32),
            pltpu.VMEM((KH, DIM), jnp.float32),
            pltpu.VMEM((DIM,), jnp.float32),
            pltpu.SemaphoreType.DMA,
        ],
    )(_sc_bag_body)


def kernel(x, W_q, ln_scale, ln_bias, keys, values):
    t, b, e = x.shape
    x2d = x.reshape(T, DIM)
    lns2d = ln_scale.reshape(1, DIM)
    lnb2d = ln_bias.reshape(1, DIM)
    kmat = jnp.transpose(keys, (2, 0, 3, 1)).reshape(2 * HEADS, DHALF, NKEYS)
    rep = (lax.broadcasted_iota(jnp.int32, (K, K * 16), 1) // 16
           == lax.broadcasted_iota(jnp.int32, (K, K * 16), 0)
           ).astype(jnp.float32)
    vi, wrep = _tc_route(x2d, W_q, lns2d, lnb2d, kmat, rep)
    out = _get_sc_bag()(vi, wrep.reshape(T, KH, 16), values)
    return out.reshape(t, b, e)


# speedup vs baseline: 12.9734x; 1.4827x over previous
import functools

import jax
import jax.numpy as jnp
from jax import lax
from jax.experimental import pallas as pl
from jax.experimental.pallas import tpu as pltpu
from jax.experimental.pallas import tpu_sc as plsc

DIM = 1024
HEADS = 4
NKEYS = 256
K = 16
DHALF = 128
T = 2048
KH = HEADS * K
TBLK = 256
NBLK = T // TBLK
NEG = float("-inf")


def _topk16_packed(d, idx_lane):
    di = lax.bitcast_convert_type(d, jnp.int32)
    dp = lax.bitcast_convert_type((di & jnp.int32(-256)) | idx_lane, jnp.float32)
    vals = []
    for _ in range(K):
        m = jnp.max(dp, axis=1, keepdims=True)
        vals.append(m)
        dp = jnp.where(dp == m, NEG, dp)
    v = jnp.concatenate(vals, axis=1)
    idx = lax.bitcast_convert_type(v, jnp.int32) & 255
    return v, idx


def _tc_body(x_ref, wq_ref, lns_ref, lnb_ref, kmat_ref, rep_ref, vi_ref, wrep_ref):
    xb = x_ref[...]
    q = jnp.dot(xb, wq_ref[...], preferred_element_type=jnp.float32,
                precision=lax.Precision.DEFAULT)
    mu = jnp.mean(q, axis=1, keepdims=True)
    var = jnp.mean((q - mu) ** 2, axis=1, keepdims=True)
    q = (q - mu) * lax.rsqrt(var + 1e-6) * lns_ref[...] + lnb_ref[...]

    dots_list = []
    for p in range(2):
        for h in range(HEADS):
            qph = q[:, p * (HEADS * DHALF) + h * DHALF:
                    p * (HEADS * DHALF) + (h + 1) * DHALF]
            dots_list.append(jnp.dot(qph, kmat_ref[p * HEADS + h],
                                     preferred_element_type=jnp.float32,
                                     precision=lax.Precision.DEFAULT))
    d1 = jnp.concatenate(dots_list, axis=0)
    iota_a = lax.broadcasted_iota(jnp.int32, (2 * HEADS * TBLK, NKEYS), 1)
    sv, si = _topk16_packed(d1, iota_a)

    comb_list = []
    for h in range(HEADS):
        s1h = sv[h * TBLK:(h + 1) * TBLK]
        s2h = sv[(HEADS + h) * TBLK:(HEADS + h + 1) * TBLK]
        comb_list.append((s1h[:, :, None] + s2h[:, None, :]).reshape(TBLK, K * K))
    comb = jnp.concatenate(comb_list, axis=0)
    iota_b = lax.broadcasted_iota(jnp.int32, (HEADS * TBLK, K * K), 1)
    fv, fc = _topk16_packed(comb, iota_b)

    a = fc >> 4
    b = fc & 15
    i1all = jnp.concatenate([si[h * TBLK:(h + 1) * TBLK] for h in range(HEADS)], axis=0)
    i2all = jnp.concatenate([si[(HEADS + h) * TBLK:(HEADS + h + 1) * TBLK]
                             for h in range(HEADS)], axis=0)
    iota16_3 = lax.broadcasted_iota(jnp.int32, (HEADS * TBLK, K, K), 2)
    i1sel = jnp.sum(jnp.where(a[:, :, None] == iota16_3, i1all[:, None, :], 0), axis=2)
    i2sel = jnp.sum(jnp.where(b[:, :, None] == iota16_3, i2all[:, None, :], 0), axis=2)
    vi = i1sel * NKEYS + i2sel
    mx = jnp.max(fv, axis=1, keepdims=True)
    e = jnp.exp(fv - mx)
    attn = e / jnp.sum(e, axis=1, keepdims=True)
    for h in range(HEADS):
        vi_ref[:, h * K:(h + 1) * K] = vi[h * TBLK:(h + 1) * TBLK]
        wrep_ref[:, h * (K * 16):(h + 1) * (K * 16)] = jnp.dot(
            attn[h * TBLK:(h + 1) * TBLK], rep_ref[...],
            preferred_element_type=jnp.float32)


def _tc_route(x2d, W_q, lns2d, lnb2d, kmat, rep):
    return pl.pallas_call(
        _tc_body,
        grid=(NBLK,),
        in_specs=[
            pl.BlockSpec((TBLK, DIM), lambda i: (i, 0)),
            pl.BlockSpec((DIM, DIM), lambda i: (0, 0)),
            pl.BlockSpec((1, DIM), lambda i: (0, 0)),
            pl.BlockSpec((1, DIM), lambda i: (0, 0)),
            pl.BlockSpec((2 * HEADS, DHALF, NKEYS), lambda i: (0, 0, 0)),
            pl.BlockSpec((K, K * 16), lambda i: (0, 0)),
        ],
        out_specs=[
            pl.BlockSpec((TBLK, KH), lambda i: (i, 0)),
            pl.BlockSpec((TBLK, KH * 16), lambda i: (i, 0)),
        ],
        out_shape=[
            jax.ShapeDtypeStruct((T, KH), jnp.int32),
            jax.ShapeDtypeStruct((T, KH * 16), jnp.float32),
        ],
    )(x2d, W_q, lns2d, lnb2d, kmat, rep)


_NC = 2
_NS = 16
_NW = _NC * _NS
_TPW = T // _NW
_U = 8
_CH = 16
_NCHK = KH // _CH
_NBUF = 4


def _sc_bag_body(vi_hbm, wrep_hbm, table_hbm, out_hbm,
                 idx_v, wr_v, acc_v,
                 rows0, rows1, rows2, rows3,
                 sem0, sem1, sem2, sem3, wsem0, wsem1):
    rows = [rows0, rows1, rows2, rows3]
    sems = [sem0, sem1, sem2, sem3]
    wsems = [wsem0, wsem1]
    steps = _TPW * _NCHK
    wid = lax.axis_index("s") * _NC + lax.axis_index("c")
    base = wid * _TPW
    pltpu.sync_copy(vi_hbm.at[pl.ds(base, _TPW)], idx_v)

    def issue(s, b):
        tl = s // _NCHK
        ci = s % _NCHK
        pltpu.make_async_copy(
            table_hbm.at[idx_v.at[tl, pl.ds(ci * _CH, _CH)]],
            rows[b], sems[b]).start()

    def wissue(tl, slot):
        pltpu.make_async_copy(
            wrep_hbm.at[base + tl],
            wr_v.at[pl.ds(slot * KH, KH)], wsems[slot]).start()

    def wwait(slot):
        pltpu.make_async_copy(
            wrep_hbm.at[base],
            wr_v.at[pl.ds(slot * KH, KH)], wsems[slot]).wait()

    for b in range(_NBUF):
        issue(b, b)
    wissue(0, 0)
    wissue(1, 1)

    def step(s, b):
        tl = s // _NCHK
        ci = s % _NCHK

        @pl.when(ci == 0)
        def _():
            @pl.when(tl % 2 == 0)
            def _():
                wwait(0)

            @pl.when(tl % 2 == 1)
            def _():
                wwait(1)

        pltpu.make_async_copy(
            table_hbm.at[idx_v.at[tl, pl.ds(ci * _CH, _CH)]],
            rows[b], sems[b]).wait()

        wbase = (tl % 2) * KH + ci * _CH

        def dgroup(g, carry2):
            def rbody(r, accs):
                wv = wr_v[wbase + r]
                return tuple(
                    accs[u] + wv * rows[b][r, pl.ds((g * _U + u) * 16, 16)]
                    for u in range(_U))
            accs = lax.fori_loop(
                0, _CH, rbody,
                tuple(jnp.zeros((16,), jnp.float32) for _ in range(_U)))

            @pl.when(ci == 0)
            def _():
                for u in range(_U):
                    acc_v[pl.ds((g * _U + u) * 16, 16)] = accs[u]

            @pl.when(ci != 0)
            def _():
                for u in range(_U):
                    plsc.addupdate(acc_v.at[pl.ds((g * _U + u) * 16, 16)],
                                   accs[u])
            return carry2
        lax.fori_loop(0, DIM // (16 * _U), dgroup, 0)

        @pl.when(ci == _NCHK - 1)
        def _():
            pltpu.sync_copy(acc_v, out_hbm.at[base + tl])

            @pl.when(tl + 2 < _TPW)
            def _():
                @pl.when(tl % 2 == 0)
                def _():
                    wissue(tl + 2, 0)

                @pl.when(tl % 2 == 1)
                def _():
                    wissue(tl + 2, 1)

        @pl.when(s + _NBUF < steps)
        def _():
            issue(s + _NBUF, b)

    def group_body(g, carry):
        for b in range(_NBUF):
            step(g * _NBUF + b, b)
        return carry
    lax.fori_loop(0, steps // _NBUF, group_body, 0)


@functools.lru_cache(maxsize=1)
def _get_sc_bag():
    return functools.partial(
        pl.kernel,
        mesh=plsc.VectorSubcoreMesh(core_axis_name="c", subcore_axis_name="s"),
        out_type=jax.ShapeDtypeStruct((T, DIM), jnp.float32),
        scratch_types=[
            pltpu.VMEM((_TPW, KH), jnp.int32),
            pltpu.VMEM((2 * KH, 16), jnp.float32),
            pltpu.VMEM((DIM,), jnp.float32),
        ] + [pltpu.VMEM((_CH, DIM), jnp.float32)] * _NBUF
          + [pltpu.SemaphoreType.DMA] * (_NBUF + 2),
    )(_sc_bag_body)


def kernel(x, W_q, ln_scale, ln_bias, keys, values):
    t, b, e = x.shape
    x2d = x.reshape(T, DIM)
    lns2d = ln_scale.reshape(1, DIM)
    lnb2d = ln_bias.reshape(1, DIM)
    kmat = jnp.transpose(keys, (2, 0, 3, 1)).reshape(2 * HEADS, DHALF, NKEYS)
    rep = (lax.broadcasted_iota(jnp.int32, (K, K * 16), 1) // 16
           == lax.broadcasted_iota(jnp.int32, (K, K * 16), 0)
           ).astype(jnp.float32)
    vi, wrep = _tc_route(x2d, W_q, lns2d, lnb2d, kmat, rep)
    out = _get_sc_bag()(vi, wrep.reshape(T, KH, 16), values)
    return out.reshape(t, b, e)
